# baseline (speedup 1.0000x reference)
import functools

import jax
import jax.numpy as jnp
from jax import lax
from jax.experimental import pallas as pl
from jax.experimental.pallas import tpu as pltpu
from jax.experimental.pallas import tpu_sc as plsc

B = 16384
EMB = 64
HID = 128
VOCAB = 1000000

NC = 2
NS = 16
NW = NC * NS
LANE_TILES = (VOCAB + 127) // 128
WAVE = 1024
NWAVES = 31
TAIL_BASE = VOCAB - (WAVE - 448)
OUT_ROWS = B + 32
JUNK_B = B + 16
ACC = 128


def _process_table(tbl_ref, tail_ref, idx_hbm, out_ref, plist, wbuf,
                   wave, acc, bacc, wsem, ssem, lo, hi):
  lane = lax.iota(jnp.int32, 16)
  junk16 = jnp.full((16,), JUNK_B, jnp.int32)

  def junk_fill_bacc():
    for k in range(ACC // 16):
      bacc[pl.ds(k * 16, 16)] = junk16

  junk_fill_bacc()

  pltpu.sync_copy(idx_hbm, wbuf.at[pl.ds(0, B)])

  def route(k, off):
    v = wbuf[pl.ds(k * 16, 16)]
    m = (v >= lo) & (v < hi)
    ci = plsc.cumsum(m.astype(jnp.int32))
    pos = jnp.where(m, off + ci - 1, OUT_ROWS - 1)
    p = ((k * 16 + lane) << 15) | (v - lo)
    plsc.store_scatter(plist, [pos], p)
    return off + jnp.max(ci)

  n = lax.fori_loop(0, B // 16, route, jnp.int32(0), unroll=2)
  plist[pl.ds(n, 16)] = jnp.full((16,), JUNK_B << 15, jnp.int32)
  nchunks = (n + 15) // 16

  def flush_issue(acc_n):
    pltpu.async_copy(acc, out_ref.at[bacc], ssem)

  def flush_wait():
    pltpu.make_async_copy(out_ref.at[pl.ds(0, ACC)], acc, ssem).wait()

  def do_wave(w, carry):
    acc_n, pending = carry
    wb = lo + w * WAVE
    use_tail = wb + WAVE > VOCAB
    wqb = jnp.where(use_tail, TAIL_BASE - lo, w * WAVE)

    @pl.when(use_tail)
    def _():
      pltpu.async_copy(tail_ref, wave, wsem).wait()

    @pl.when(jnp.logical_not(use_tail))
    def _():
      pltpu.async_copy(
          tbl_ref.at[:, pl.ds(pl.multiple_of(wb, 128), WAVE)], wave, wsem
      ).wait()

    def scan(k, nw):
      p = plist[pl.ds(k * 16, 16)]
      c = (p & 0x7FFF) - wqb
      m = (c >= 0) & (c < WAVE)
      ci = plsc.cumsum(m.astype(jnp.int32))
      pos = jnp.where(m, nw + ci - 1, OUT_ROWS - 1)
      q = ((p >> 15) << 10) | jnp.where(m, c, 0)
      plsc.store_scatter(wbuf, [pos], q)
      return nw + jnp.max(ci)

    nw = lax.fori_loop(0, nchunks, scan, jnp.int32(0))
    wbuf[pl.ds(nw, 16)] = jnp.full((16,), JUNK_B << 10, jnp.int32)

    @pl.when(pending > 0)
    def _():
      flush_wait()
      junk_fill_bacc()

    ngrp = jnp.where(nw > 0, (nw + 15) // 16, 0)

    def grp(g, acc_n_):
      @pl.when(acc_n_ + 16 > ACC)
      def _():
        flush_issue(acc_n_)
        flush_wait()
        junk_fill_bacc()

      acc_n_ = jnp.where(acc_n_ + 16 > ACC, 0, acc_n_)
      q = wbuf[pl.ds(g * 16, 16)]
      c16 = q & 1023
      b16 = q >> 10
      ord16 = acc_n_ + lane

      def eloop(e8, _2):
        for ee in range(8):
          e = e8 * 8 + ee
          sp = jnp.full((16,), 1, jnp.int32) * e
          vals = plsc.load_gather(wave, [sp, c16])
          plsc.store_scatter(acc, [ord16, sp], vals)
        return _2

      lax.fori_loop(0, 8, eloop, jnp.int32(0))
      bacc[pl.ds(acc_n_, 16)] = b16
      return acc_n_ + 16

    acc_n = lax.fori_loop(0, ngrp, grp, acc_n)

    do_flush = acc_n > ACC - 16
    @pl.when(do_flush)
    def _():
      flush_issue(acc_n)

    acc_n = jnp.where(do_flush, 0, acc_n)
    return (acc_n, jnp.where(do_flush, 1, 0).astype(jnp.int32))

  acc_n, pending = lax.fori_loop(
      0, NWAVES, do_wave, (jnp.int32(0), jnp.int32(0)))

  @pl.when(pending > 0)
  def _():
    flush_wait()
    junk_fill_bacc()

  @pl.when(acc_n > 0)
  def _():
    flush_issue(acc_n)
    flush_wait()
    junk_fill_bacc()


def _sc_gather(uid, iid, ut_t, it_t, u_tail, i_tail):
  mesh = plsc.VectorSubcoreMesh(core_axis_name="c", subcore_axis_name="s")

  @functools.partial(
      pl.kernel,
      mesh=mesh,
      compiler_params=pltpu.CompilerParams(needs_layout_passes=False),
      out_type=[
          jax.ShapeDtypeStruct((OUT_ROWS, HID), jnp.float32),
          jax.ShapeDtypeStruct((OUT_ROWS, HID), jnp.float32),
      ],
      scratch_types=[
          pltpu.VMEM((OUT_ROWS,), jnp.int32),
          pltpu.VMEM((OUT_ROWS,), jnp.int32),
          pltpu.VMEM((EMB, WAVE), jnp.float32),
          pltpu.VMEM((ACC, HID), jnp.float32),
          pltpu.VMEM((ACC,), jnp.int32),
          pltpu.SemaphoreType.DMA,
          pltpu.SemaphoreType.DMA,
      ],
  )
  def gather_kernel(uid_hbm, iid_hbm, ut_hbm, it_hbm, ut_tail, it_tail,
                    u_out, i_out, plist, wbuf, wave, acc, bacc, wsem, ssem):
    wid = lax.axis_index("s") * NC + lax.axis_index("c")
    lo = ((wid * LANE_TILES) // NW) * 128
    hi = (((wid + 1) * LANE_TILES) // NW) * 128
    _process_table(ut_hbm, ut_tail, uid_hbm, u_out, plist, wbuf,
                   wave, acc, bacc, wsem, ssem, lo, hi)
    _process_table(it_hbm, it_tail, iid_hbm, i_out, plist, wbuf,
                   wave, acc, bacc, wsem, ssem, lo, hi)

  return gather_kernel(uid, iid, ut_t, it_t, u_tail, i_tail)


def _dense_body(u_ref, i_ref, wu_ref, bu_ref, wi_ref, bi_ref, o_ref):
  dn = (((1,), (0,)), ((), ()))
  u = u_ref[...][:, :EMB]
  i = i_ref[...][:, :EMB]
  uf = lax.dot_general(u, wu_ref[...], dn, preferred_element_type=jnp.float32)
  uf = jnp.maximum(uf + bu_ref[...], 0.0)
  itf = lax.dot_general(i, wi_ref[...], dn, preferred_element_type=jnp.float32)
  itf = jnp.maximum(itf + bi_ref[...], 0.0)
  o_ref[...] = jnp.sum(uf * itf, axis=1)


def _tc_dense(u_emb, i_emb, W_u, b_u, W_i, b_i):
  nb = 2048
  grid = B // nb
  return pl.pallas_call(
      _dense_body,
      grid=(grid,),
      in_specs=[
          pl.BlockSpec((nb, HID), lambda b: (b, 0)),
          pl.BlockSpec((nb, HID), lambda b: (b, 0)),
          pl.BlockSpec((EMB, HID), lambda b: (0, 0)),
          pl.BlockSpec((1, HID), lambda b: (0, 0)),
          pl.BlockSpec((EMB, HID), lambda b: (0, 0)),
          pl.BlockSpec((1, HID), lambda b: (0, 0)),
      ],
      out_specs=pl.BlockSpec((nb,), lambda b: (b,)),
      out_shape=jax.ShapeDtypeStruct((B,), jnp.float32),
  )(u_emb, i_emb, W_u, b_u.reshape(1, HID), W_i, b_i.reshape(1, HID))


def kernel(uid_batch, iid_batch, user_table, item_table, W_u, b_u, W_i, b_i):
  ut_t = user_table.T
  it_t = item_table.T
  u_tail = jnp.pad(ut_t[:, TAIL_BASE:], ((0, 0), (0, WAVE - (VOCAB - TAIL_BASE))))
  i_tail = jnp.pad(it_t[:, TAIL_BASE:], ((0, 0), (0, WAVE - (VOCAB - TAIL_BASE))))
  u_emb, i_emb = _sc_gather(uid_batch, iid_batch, ut_t, it_t, u_tail, i_tail)
  return _tc_dense(u_emb, i_emb, W_u, b_u, W_i, b_i)


# speedup vs baseline: 3.3768x; 3.3768x over previous
import functools

import jax
import jax.numpy as jnp
from jax import lax
from jax.experimental import pallas as pl
from jax.experimental.pallas import tpu as pltpu
from jax.experimental.pallas import tpu_sc as plsc

B = 16384
EMB = 64
HID = 128
VOCAB = 1000000

NC = 2
NS = 16
NW = NC * NS
LANE_TILES = (VOCAB + 127) // 128
WAVE = 1024
NWAVES = 31
TAIL_BASE = VOCAB - (WAVE - 448)
PBUF_ROWS = B + 32
OUT_ROWS = B + NW * 128
ACC = 128


def _process_table(tbl_ref, tail_ref, idx_hbm, out_ref, plist, wbuf,
                   wave, acc, bacc, wsem, ssem, lo, hi, jb):
  lane = lax.iota(jnp.int32, 16)

  def junk_fill_bacc():
    for k in range(ACC // 16):
      bacc[pl.ds(k * 16, 16)] = jb + k * 16 + lane

  junk_fill_bacc()

  pltpu.sync_copy(idx_hbm, wbuf.at[pl.ds(0, B)])

  def route(k, off):
    v = wbuf[pl.ds(k * 16, 16)]
    m = (v >= lo) & (v < hi)
    ci = plsc.cumsum(m.astype(jnp.int32))
    pos = jnp.where(m, off + ci - 1, PBUF_ROWS - 1)
    p = ((k * 16 + lane) << 15) | (v - lo)
    plsc.store_scatter(plist, [pos], p)
    return off + jnp.max(ci)

  n = lax.fori_loop(0, B // 16, route, jnp.int32(0), unroll=2)
  plist[pl.ds(n, 16)] = (jb + lane) << 15
  nchunks = (n + 15) // 16

  def flush_issue(acc_n):
    pltpu.async_copy(acc, out_ref.at[bacc], ssem)

  def flush_wait():
    pltpu.make_async_copy(out_ref.at[pl.ds(0, ACC)], acc, ssem).wait()

  def do_wave(w, carry):
    acc_n, pending = carry
    wb = lo + w * WAVE
    use_tail = wb + WAVE > VOCAB
    wqb = jnp.where(use_tail, TAIL_BASE - lo, w * WAVE)

    @pl.when(use_tail)
    def _():
      pltpu.async_copy(tail_ref, wave, wsem).wait()

    @pl.when(jnp.logical_not(use_tail))
    def _():
      pltpu.async_copy(
          tbl_ref.at[:, pl.ds(pl.multiple_of(wb, 128), WAVE)], wave, wsem
      ).wait()

    def scan(k, nw):
      p = plist[pl.ds(k * 16, 16)]
      c = (p & 0x7FFF) - wqb
      m = (c >= 0) & (c < WAVE)
      ci = plsc.cumsum(m.astype(jnp.int32))
      pos = jnp.where(m, nw + ci - 1, PBUF_ROWS - 1)
      q = ((p >> 15) << 10) | jnp.where(m, c, 0)
      plsc.store_scatter(wbuf, [pos], q)
      return nw + jnp.max(ci)

    nw = lax.fori_loop(0, nchunks, scan, jnp.int32(0))
    wbuf[pl.ds(nw, 16)] = (jb + lane) << 10

    @pl.when(pending > 0)
    def _():
      flush_wait()
      junk_fill_bacc()

    ngrp = jnp.where(nw > 0, (nw + 15) // 16, 0)

    def grp(g, acc_n_):
      @pl.when(acc_n_ + 16 > ACC)
      def _():
        flush_issue(acc_n_)
        flush_wait()
        junk_fill_bacc()

      acc_n_ = jnp.where(acc_n_ + 16 > ACC, 0, acc_n_)
      q = wbuf[pl.ds(g * 16, 16)]
      c16 = q & 1023
      b16 = q >> 10
      ord16 = acc_n_ + lane

      def eloop(e8, _2):
        for ee in range(8):
          e = e8 * 8 + ee
          sp = jnp.full((16,), 1, jnp.int32) * e
          vals = plsc.load_gather(wave, [sp, c16])
          plsc.store_scatter(acc, [ord16, sp], vals)
        return _2

      lax.fori_loop(0, 8, eloop, jnp.int32(0))
      bacc[pl.ds(acc_n_, 16)] = b16
      return acc_n_ + 16

    acc_n = lax.fori_loop(0, ngrp, grp, acc_n)

    do_flush = acc_n > ACC - 16
    @pl.when(do_flush)
    def _():
      flush_issue(acc_n)

    acc_n = jnp.where(do_flush, 0, acc_n)
    return (acc_n, jnp.where(do_flush, 1, 0).astype(jnp.int32))

  acc_n, pending = lax.fori_loop(
      0, NWAVES, do_wave, (jnp.int32(0), jnp.int32(0)))

  @pl.when(pending > 0)
  def _():
    flush_wait()
    junk_fill_bacc()

  @pl.when(acc_n > 0)
  def _():
    flush_issue(acc_n)
    flush_wait()
    junk_fill_bacc()


def _sc_gather(uid, iid, ut_t, it_t, u_tail, i_tail):
  mesh = plsc.VectorSubcoreMesh(core_axis_name="c", subcore_axis_name="s")

  @functools.partial(
      pl.kernel,
      mesh=mesh,
      compiler_params=pltpu.CompilerParams(needs_layout_passes=False),
      out_type=[
          jax.ShapeDtypeStruct((OUT_ROWS, HID), jnp.float32),
          jax.ShapeDtypeStruct((OUT_ROWS, HID), jnp.float32),
      ],
      scratch_types=[
          pltpu.VMEM((PBUF_ROWS,), jnp.int32),
          pltpu.VMEM((PBUF_ROWS,), jnp.int32),
          pltpu.VMEM((EMB, WAVE), jnp.float32),
          pltpu.VMEM((ACC, HID), jnp.float32),
          pltpu.VMEM((ACC,), jnp.int32),
          pltpu.SemaphoreType.DMA,
          pltpu.SemaphoreType.DMA,
      ],
  )
  def gather_kernel(uid_hbm, iid_hbm, ut_hbm, it_hbm, ut_tail, it_tail,
                    u_out, i_out, plist, wbuf, wave, acc, bacc, wsem, ssem):
    wid = lax.axis_index("s") * NC + lax.axis_index("c")
    lo = ((wid * LANE_TILES) // NW) * 128
    hi = (((wid + 1) * LANE_TILES) // NW) * 128
    jb = B + wid * 128
    _process_table(ut_hbm, ut_tail, uid_hbm, u_out, plist, wbuf,
                   wave, acc, bacc, wsem, ssem, lo, hi, jb)
    _process_table(it_hbm, it_tail, iid_hbm, i_out, plist, wbuf,
                   wave, acc, bacc, wsem, ssem, lo, hi, jb)

  return gather_kernel(uid, iid, ut_t, it_t, u_tail, i_tail)


def _dense_body(u_ref, i_ref, wu_ref, bu_ref, wi_ref, bi_ref, o_ref):
  dn = (((1,), (0,)), ((), ()))
  u = u_ref[...][:, :EMB]
  i = i_ref[...][:, :EMB]
  uf = lax.dot_general(u, wu_ref[...], dn, preferred_element_type=jnp.float32)
  uf = jnp.maximum(uf + bu_ref[...], 0.0)
  itf = lax.dot_general(i, wi_ref[...], dn, preferred_element_type=jnp.float32)
  itf = jnp.maximum(itf + bi_ref[...], 0.0)
  o_ref[...] = jnp.sum(uf * itf, axis=1)


def _tc_dense(u_emb, i_emb, W_u, b_u, W_i, b_i):
  nb = 2048
  grid = B // nb
  return pl.pallas_call(
      _dense_body,
      grid=(grid,),
      in_specs=[
          pl.BlockSpec((nb, HID), lambda b: (b, 0)),
          pl.BlockSpec((nb, HID), lambda b: (b, 0)),
          pl.BlockSpec((EMB, HID), lambda b: (0, 0)),
          pl.BlockSpec((1, HID), lambda b: (0, 0)),
          pl.BlockSpec((EMB, HID), lambda b: (0, 0)),
          pl.BlockSpec((1, HID), lambda b: (0, 0)),
      ],
      out_specs=pl.BlockSpec((nb,), lambda b: (b,)),
      out_shape=jax.ShapeDtypeStruct((B,), jnp.float32),
  )(u_emb, i_emb, W_u, b_u.reshape(1, HID), W_i, b_i.reshape(1, HID))


def kernel(uid_batch, iid_batch, user_table, item_table, W_u, b_u, W_i, b_i):
  ut_t = user_table.T
  it_t = item_table.T
  u_tail = jnp.pad(ut_t[:, TAIL_BASE:], ((0, 0), (0, WAVE - (VOCAB - TAIL_BASE))))
  i_tail = jnp.pad(it_t[:, TAIL_BASE:], ((0, 0), (0, WAVE - (VOCAB - TAIL_BASE))))
  u_emb, i_emb = _sc_gather(uid_batch, iid_batch, ut_t, it_t, u_tail, i_tail)
  return _tc_dense(u_emb, i_emb, W_u, b_u, W_i, b_i)


# speedup vs baseline: 4.0756x; 1.2069x over previous
import functools

import jax
import jax.numpy as jnp
from jax import lax
from jax.experimental import pallas as pl
from jax.experimental.pallas import tpu as pltpu
from jax.experimental.pallas import tpu_sc as plsc

B = 16384
EMB = 64
HID = 128
VOCAB = 1000000

NC = 2
NS = 16
NW = NC * NS
LANE_TILES = (VOCAB + 127) // 128
WAVE = 512
NWAVES = 62
NPAIRS = NWAVES // 2
TAIL_BASE = VOCAB - WAVE
PBUF_ROWS = B + 32
OUT_ROWS = B + NW * 128
ACC = 128


def _process_table(tbl_ref, tail_ref, idx_hbm, out_ref, plist, wbuf,
                   wave0, wave1, acc, bacc, wsem0, wsem1, ssem, lo, hi, jb):
  lane = lax.iota(jnp.int32, 16)

  def junk_fill_bacc():
    for k in range(ACC // 16):
      bacc[pl.ds(k * 16, 16)] = jb + k * 16 + lane

  junk_fill_bacc()

  pltpu.sync_copy(idx_hbm, wbuf.at[pl.ds(0, B)])

  def route(k, off):
    v = wbuf[pl.ds(k * 16, 16)]
    m = (v >= lo) & (v < hi)
    ci = plsc.cumsum(m.astype(jnp.int32))
    pos = jnp.where(m, off + ci - 1, PBUF_ROWS - 1)
    p = ((k * 16 + lane) << 15) | (v - lo)
    plsc.store_scatter(plist, [pos], p)
    return off + jnp.max(ci)

  n = lax.fori_loop(0, B // 16, route, jnp.int32(0), unroll=2)
  plist[pl.ds(n, 16)] = (jb + lane) << 15
  nchunks = (n + 15) // 16

  def flush_issue(acc_n):
    pltpu.async_copy(acc, out_ref.at[bacc], ssem)

  def flush_wait():
    pltpu.make_async_copy(out_ref.at[pl.ds(0, ACC)], acc, ssem).wait()

  def issue(w, buf, sem):
    wc = jnp.minimum(w, NWAVES - 1)
    gb = lo + wc * WAVE
    use_tail = gb + WAVE > VOCAB

    @pl.when(use_tail)
    def _():
      pltpu.async_copy(tail_ref, buf, sem)

    @pl.when(jnp.logical_not(use_tail))
    def _():
      pltpu.async_copy(
          tbl_ref.at[:, pl.ds(pl.multiple_of(gb, 128), WAVE)], buf, sem)

  def process(w, wave, wsem, carry):
    acc_n, pending = carry
    wb = lo + w * WAVE
    use_tail = wb + WAVE > VOCAB
    wqb = jnp.where(use_tail, TAIL_BASE - lo, w * WAVE)

    pltpu.make_async_copy(tbl_ref.at[:, pl.ds(0, WAVE)], wave, wsem).wait()

    def scan(k, nw):
      p = plist[pl.ds(k * 16, 16)]
      c = (p & 0x7FFF) - wqb
      m = (c >= 0) & (c < WAVE)
      ci = plsc.cumsum(m.astype(jnp.int32))
      pos = jnp.where(m, nw + ci - 1, PBUF_ROWS - 1)
      q = ((p >> 15) << 10) | jnp.where(m, c, 0)
      plsc.store_scatter(wbuf, [pos], q)
      return nw + jnp.max(ci)

    nw = lax.fori_loop(0, nchunks, scan, jnp.int32(0))
    wbuf[pl.ds(nw, 16)] = (jb + lane) << 10

    @pl.when(pending > 0)
    def _():
      flush_wait()
      junk_fill_bacc()

    ngrp = jnp.where(nw > 0, (nw + 15) // 16, 0)

    def grp(g, acc_n_):
      @pl.when(acc_n_ + 16 > ACC)
      def _():
        flush_issue(acc_n_)
        flush_wait()
        junk_fill_bacc()

      acc_n_ = jnp.where(acc_n_ + 16 > ACC, 0, acc_n_)
      q = wbuf[pl.ds(g * 16, 16)]
      c16 = q & 1023
      b16 = q >> 10
      ord16 = acc_n_ + lane

      def eloop(e8, _2):
        for ee in range(8):
          e = e8 * 8 + ee
          sp = jnp.full((16,), 1, jnp.int32) * e
          vals = plsc.load_gather(wave, [sp, c16])
          plsc.store_scatter(acc, [ord16, sp], vals)
        return _2

      lax.fori_loop(0, 8, eloop, jnp.int32(0))
      bacc[pl.ds(acc_n_, 16)] = b16
      return acc_n_ + 16

    acc_n = lax.fori_loop(0, ngrp, grp, acc_n)

    do_flush = acc_n > ACC - 16
    @pl.when(do_flush)
    def _():
      flush_issue(acc_n)

    acc_n = jnp.where(do_flush, 0, acc_n)
    return (acc_n, jnp.where(do_flush, 1, 0).astype(jnp.int32))

  issue(0, wave0, wsem0)

  def pair(j, carry):
    w0 = j * 2
    issue(w0 + 1, wave1, wsem1)
    carry = process(w0, wave0, wsem0, carry)
    issue(w0 + 2, wave0, wsem0)
    carry = process(w0 + 1, wave1, wsem1, carry)
    return carry

  acc_n, pending = lax.fori_loop(
      0, NPAIRS, pair, (jnp.int32(0), jnp.int32(0)))
  pltpu.make_async_copy(tbl_ref.at[:, pl.ds(0, WAVE)], wave0, wsem0).wait()

  @pl.when(pending > 0)
  def _():
    flush_wait()
    junk_fill_bacc()

  @pl.when(acc_n > 0)
  def _():
    flush_issue(acc_n)
    flush_wait()
    junk_fill_bacc()


def _sc_gather(uid, iid, ut_t, it_t, u_tail, i_tail):
  mesh = plsc.VectorSubcoreMesh(core_axis_name="c", subcore_axis_name="s")

  @functools.partial(
      pl.kernel,
      mesh=mesh,
      compiler_params=pltpu.CompilerParams(needs_layout_passes=False),
      out_type=[
          jax.ShapeDtypeStruct((OUT_ROWS, HID), jnp.float32),
          jax.ShapeDtypeStruct((OUT_ROWS, HID), jnp.float32),
      ],
      scratch_types=[
          pltpu.VMEM((PBUF_ROWS,), jnp.int32),
          pltpu.VMEM((PBUF_ROWS,), jnp.int32),
          pltpu.VMEM((EMB, WAVE), jnp.float32),
          pltpu.VMEM((EMB, WAVE), jnp.float32),
          pltpu.VMEM((ACC, HID), jnp.float32),
          pltpu.VMEM((ACC,), jnp.int32),
          pltpu.SemaphoreType.DMA,
          pltpu.SemaphoreType.DMA,
          pltpu.SemaphoreType.DMA,
      ],
  )
  def gather_kernel(uid_hbm, iid_hbm, ut_hbm, it_hbm, ut_tail, it_tail,
                    u_out, i_out, plist, wbuf, wave0, wave1, acc, bacc,
                    wsem0, wsem1, ssem):
    wid = lax.axis_index("s") * NC + lax.axis_index("c")
    lo = ((wid * LANE_TILES) // NW) * 128
    hi = (((wid + 1) * LANE_TILES) // NW) * 128
    jb = B + wid * 128
    _process_table(ut_hbm, ut_tail, uid_hbm, u_out, plist, wbuf,
                   wave0, wave1, acc, bacc, wsem0, wsem1, ssem, lo, hi, jb)
    _process_table(it_hbm, it_tail, iid_hbm, i_out, plist, wbuf,
                   wave0, wave1, acc, bacc, wsem0, wsem1, ssem, lo, hi, jb)

  return gather_kernel(uid, iid, ut_t, it_t, u_tail, i_tail)


def _dense_body(u_ref, i_ref, wu_ref, bu_ref, wi_ref, bi_ref, o_ref):
  dn = (((1,), (0,)), ((), ()))
  u = u_ref[...][:, :EMB]
  i = i_ref[...][:, :EMB]
  uf = lax.dot_general(u, wu_ref[...], dn, preferred_element_type=jnp.float32)
  uf = jnp.maximum(uf + bu_ref[...], 0.0)
  itf = lax.dot_general(i, wi_ref[...], dn, preferred_element_type=jnp.float32)
  itf = jnp.maximum(itf + bi_ref[...], 0.0)
  o_ref[...] = jnp.sum(uf * itf, axis=1)


def _tc_dense(u_emb, i_emb, W_u, b_u, W_i, b_i):
  nb = 2048
  grid = B // nb
  return pl.pallas_call(
      _dense_body,
      grid=(grid,),
      in_specs=[
          pl.BlockSpec((nb, HID), lambda b: (b, 0)),
          pl.BlockSpec((nb, HID), lambda b: (b, 0)),
          pl.BlockSpec((EMB, HID), lambda b: (0, 0)),
          pl.BlockSpec((1, HID), lambda b: (0, 0)),
          pl.BlockSpec((EMB, HID), lambda b: (0, 0)),
          pl.BlockSpec((1, HID), lambda b: (0, 0)),
      ],
      out_specs=pl.BlockSpec((nb,), lambda b: (b,)),
      out_shape=jax.ShapeDtypeStruct((B,), jnp.float32),
  )(u_emb, i_emb, W_u, b_u.reshape(1, HID), W_i, b_i.reshape(1, HID))


def kernel(uid_batch, iid_batch, user_table, item_table, W_u, b_u, W_i, b_i):
  ut_t = user_table.T
  it_t = item_table.T
  u_tail = jnp.pad(ut_t[:, TAIL_BASE:], ((0, 0), (0, WAVE - (VOCAB - TAIL_BASE))))
  i_tail = jnp.pad(it_t[:, TAIL_BASE:], ((0, 0), (0, WAVE - (VOCAB - TAIL_BASE))))
  u_emb, i_emb = _sc_gather(uid_batch, iid_batch, ut_t, it_t, u_tail, i_tail)
  return _tc_dense(u_emb, i_emb, W_u, b_u, W_i, b_i)
